# Initial kernel scaffold; baseline (speedup 1.0000x reference)
import jax
import jax.numpy as jnp
from jax.experimental import pallas as pl


def kernel(values, indices, idx_identity, idx_transpose, W0, b0, W1, b1, W2, b2, Wp, bp):
    raise NotImplementedError("write your pallas kernel here")


# speedup vs baseline: 2.9579x; 2.9579x over previous
import functools

import jax
import jax.numpy as jnp
from jax import lax
from jax.experimental import pallas as pl
from jax.experimental.pallas import tpu as pltpu
from jax.experimental.pallas import tpu_sc as plsc

NN = 50000
NNZ = 800000
NC, NS, LANES = 2, 16, 16
NW = NC * NS

_MESH = dict(core_axis_name="c", subcore_axis_name="s")
_NOTILE = pltpu.CompilerParams(use_tc_tiling_on_sc=False)


def _pick_chunk(epw, row_bytes, budget):
    best = 8
    for d in range(8, epw + 1, 8):
        if epw % d == 0 and d * row_bytes <= budget:
            best = d
    return best


def _tc_matmul(terms, cnt=None, const=None, n_out=1, blk=5000):
    M = terms[0][0].shape[0]
    N = terms[0][1].shape[1]
    assert M % blk == 0
    nt = len(terms)
    hasc = cnt is not None
    hask = const is not None

    def body(*refs):
        i = 2 * nt
        cnt_ref = k_ref = None
        if hasc:
            cnt_ref = refs[i]; i += 1
        if hask:
            k_ref = refs[i]; i += 1
        acc = None
        inv = None
        if hasc:
            inv = 1.0 / jnp.maximum(cnt_ref[...], 1.0)
        for t in range(nt):
            x = refs[2 * t][...]
            if terms[t][2]:
                x = x * inv
            p = jnp.dot(x, refs[2 * t + 1][...],
                        preferred_element_type=jnp.float32)
            acc = p if acc is None else acc + p
        if hask:
            acc = acc + k_ref[...]
        h = N // n_out
        for o in range(n_out):
            refs[i + o][...] = acc[:, o * h:(o + 1) * h]

    in_specs = []
    args = []
    for (x, w, _) in terms:
        K = x.shape[1]
        in_specs += [pl.BlockSpec((blk, K), lambda i: (i, 0)),
                     pl.BlockSpec((K, N), lambda i: (0, 0))]
        args += [x, w]
    if hasc:
        in_specs += [pl.BlockSpec((blk, 1), lambda i: (i, 0))]
        args += [cnt]
    if hask:
        in_specs += [pl.BlockSpec((1, N), lambda i: (0, 0))]
        args += [const]

    h = N // n_out
    out_specs = [pl.BlockSpec((blk, h), lambda i: (i, 0))] * n_out
    out_shape = [jax.ShapeDtypeStruct((M, h), jnp.float32)] * n_out
    if n_out == 1:
        out_specs, out_shape = out_specs[0], out_shape[0]

    return pl.pallas_call(
        body,
        grid=(M // blk,),
        in_specs=in_specs,
        out_specs=out_specs,
        out_shape=out_shape,
        compiler_params=pltpu.CompilerParams(
            vmem_limit_bytes=100 * 1024 * 1024),
    )(*args)


def _sc_gather(table, idx):
    T, C = table.shape
    M = idx.shape[0]
    assert M % (8 * NW) == 0
    epw = M // NW
    ch = _pick_chunk(epw, C * 4, 256 * 1024)
    nch = epw // ch

    @functools.partial(
        pl.kernel,
        out_type=jax.ShapeDtypeStruct((M, C), jnp.float32),
        mesh=plsc.VectorSubcoreMesh(**_MESH),
        compiler_params=_NOTILE,
        scratch_types=[
            pltpu.VMEM((ch,), jnp.int32),
            pltpu.VMEM((ch, C), jnp.float32),
            pltpu.SemaphoreType.DMA,
        ],
    )
    def k(tab_h, idx_h, out_h, idx_v, rows_v, sem):
        wid = lax.axis_index("s") * NC + lax.axis_index("c")
        base0 = wid * epw

        def chunk(i, carry):
            base = base0 + i * ch
            pltpu.sync_copy(idx_h.at[pl.ds(base, ch)], idx_v)
            pltpu.async_copy(tab_h.at[idx_v], rows_v, sem).wait()
            pltpu.sync_copy(rows_v, out_h.at[pl.ds(base, ch)])
            return carry

        lax.fori_loop(0, nch, chunk, 0)

    return k(table, idx)


def _flush_ranges(body):
    s = lax.axis_index("s")

    @pl.when(s < 15)
    def _():
        body(s * 3128, 3128)

    @pl.when(s == 15)
    def _():
        body(15 * 3128, NN - 15 * 3128)


def _load_either_idx(row_h, col_h, base, ch, idx_v):
    c = lax.axis_index("c")

    @pl.when(c == 0)
    def _():
        pltpu.sync_copy(row_h.at[pl.ds(base, ch)], idx_v)

    @pl.when(c == 1)
    def _():
        pltpu.sync_copy(col_h.at[pl.ds(base, ch)], idx_v)


def _sc_scatter16(a, row, col, zeros):
    M, C = a.shape
    assert C == 16
    ept = M // NS
    ch = _pick_chunk(ept, C * 4, 192 * 1024)
    nch = ept // ch

    @functools.partial(
        pl.kernel,
        out_type=jax.ShapeDtypeStruct((2, NN, C), jnp.float32),
        mesh=plsc.VectorSubcoreMesh(**_MESH),
        compiler_params=_NOTILE,
        scratch_types=[
            pltpu.VMEM((ch,), jnp.int32),
            pltpu.VMEM((ch, C), jnp.float32),
            pltpu.VMEM_SHARED((NN, C), jnp.float32),
        ],
    )
    def k(a_h, row_h, col_h, z_h, out_h, idx_v, data_v, accum):
        c = lax.axis_index("c")
        s = lax.axis_index("s")
        _flush_ranges(lambda b, n: pltpu.sync_copy(z_h.at[pl.ds(b, n)],
                                                   accum.at[pl.ds(b, n)]))
        plsc.subcore_barrier()

        def chunk(i, carry):
            base = s * ept + i * ch
            _load_either_idx(row_h, col_h, base, ch, idx_v)
            pltpu.sync_copy(a_h.at[pl.ds(base, ch)], data_v)
            pltpu.sync_copy(data_v, accum.at[idx_v], add=True)
            return carry

        lax.fori_loop(0, nch, chunk, 0)
        plsc.subcore_barrier()
        _flush_ranges(lambda b, n: pltpu.sync_copy(accum.at[pl.ds(b, n)],
                                                   out_h.at[c, pl.ds(b, n)]))

    return k(a, row, col, zeros)


def _sc_scatter_ones(row, col, zeros):
    M = row.shape[0]
    ept = M // NS
    ch = _pick_chunk(ept, 16 * 4, 192 * 1024)
    nch = ept // ch

    @functools.partial(
        pl.kernel,
        out_type=jax.ShapeDtypeStruct((2, NN, 16), jnp.float32),
        mesh=plsc.VectorSubcoreMesh(**_MESH),
        compiler_params=_NOTILE,
        scratch_types=[
            pltpu.VMEM((ch,), jnp.int32),
            pltpu.VMEM((ch, 16), jnp.float32),
            pltpu.VMEM_SHARED((NN, 16), jnp.float32),
        ],
    )
    def k(row_h, col_h, z_h, out_h, idx_v, ones_v, accum):
        c = lax.axis_index("c")
        s = lax.axis_index("s")
        one = jnp.full((16,), 1.0, jnp.float32)

        def fill(i, carry):
            ones_v[i, :] = one
            return carry

        lax.fori_loop(0, ch, fill, 0)
        _flush_ranges(lambda b, n: pltpu.sync_copy(z_h.at[pl.ds(b, n)],
                                                   accum.at[pl.ds(b, n)]))
        plsc.subcore_barrier()

        def chunk(i, carry):
            base = s * ept + i * ch
            _load_either_idx(row_h, col_h, base, ch, idx_v)
            pltpu.sync_copy(ones_v, accum.at[idx_v], add=True)
            return carry

        lax.fori_loop(0, nch, chunk, 0)
        plsc.subcore_barrier()
        _flush_ranges(lambda b, n: pltpu.sync_copy(accum.at[pl.ds(b, n)],
                                                   out_h.at[c, pl.ds(b, n)]))

    return k(row, col, zeros)


def _sc_combine(p_blocks, rtab, ctab, row, col, const):
    nb = len(p_blocks)
    M = p_blocks[0].shape[0]
    C = rtab.shape[1]
    assert C == 16 * nb
    epw = M // NW
    ch = _pick_chunk(epw, 3 * C * 4, 400 * 1024)
    nch = epw // ch

    out_type = tuple([jax.ShapeDtypeStruct((M, 16), jnp.float32)] * nb
                     + [jax.ShapeDtypeStruct((NW, 1, 2 * C), jnp.float32)])
    scratch = ([pltpu.VMEM((ch,), jnp.int32), pltpu.VMEM((ch,), jnp.int32)]
               + [pltpu.VMEM((ch, 16), jnp.float32) for _ in range(nb)]
               + [pltpu.VMEM((ch, C), jnp.float32),
                  pltpu.VMEM((ch, C), jnp.float32),
                  pltpu.VMEM((C,), jnp.float32),
                  pltpu.VMEM((1, 2 * C), jnp.float32),
                  pltpu.SemaphoreType.DMA])

    @functools.partial(
        pl.kernel,
        out_type=out_type,
        mesh=plsc.VectorSubcoreMesh(**_MESH),
        compiler_params=_NOTILE,
        scratch_types=scratch,
    )
    def k(*refs):
        i = 0
        p_h = refs[i:i + nb]; i += nb
        r_h = refs[i]; i += 1
        c_h = refs[i]; i += 1
        row_h = refs[i]; i += 1
        col_h = refs[i]; i += 1
        const_h = refs[i]; i += 1
        a_h = refs[i:i + nb]; i += nb
        part_h = refs[i]; i += 1
        ri = refs[i]; i += 1
        ci = refs[i]; i += 1
        p_v = refs[i:i + nb]; i += nb
        rg = refs[i]; i += 1
        cg = refs[i]; i += 1
        const_v = refs[i]; i += 1
        part_v = refs[i]; i += 1
        sem = refs[i]

        wid = lax.axis_index("s") * NC + lax.axis_index("c")
        base0 = wid * epw
        pltpu.sync_copy(const_h, const_v)
        cvecs = [const_v[pl.ds(cb * 16, 16)] for cb in range(nb)]
        zero = jnp.zeros((16,), jnp.float32)

        def chunk(i, sums):
            base = base0 + i * ch
            pltpu.sync_copy(row_h.at[pl.ds(base, ch)], ri)
            pltpu.sync_copy(col_h.at[pl.ds(base, ch)], ci)
            for b in range(nb):
                pltpu.sync_copy(p_h[b].at[pl.ds(base, ch)], p_v[b])
            pltpu.async_copy(r_h.at[ri], rg, sem).wait()
            pltpu.async_copy(c_h.at[ci], cg, sem).wait()

            def rowloop(j, sums):
                out = []
                for cb in range(nb):
                    sl = pl.ds(cb * 16, 16)
                    x = p_v[cb][j, :] + rg[j, sl] + cg[j, sl] + cvecs[cb]
                    x = jnp.maximum(x, 0.0)
                    p_v[cb][j, :] = x
                    out.append(sums[2 * cb] + x)
                    out.append(sums[2 * cb + 1] + x * x)
                return tuple(out)

            sums = lax.fori_loop(0, ch, rowloop, sums)
            for b in range(nb):
                pltpu.sync_copy(p_v[b], a_h[b].at[pl.ds(base, ch)])
            return sums

        sums = lax.fori_loop(0, nch, chunk,
                             tuple(zero for _ in range(2 * nb)))
        for cb in range(nb):
            part_v[0, pl.ds(cb * 16, 16)] = sums[2 * cb]
            part_v[0, pl.ds(C + cb * 16, 16)] = sums[2 * cb + 1]
        pltpu.sync_copy(part_v, part_h.at[wid])

    res = k(*p_blocks, rtab, ctab, row, col, const)
    return list(res[:nb]), res[nb]


def kernel(values, indices, idx_identity, idx_transpose,
           W0, b0, W1, b1, W2, b2, Wp, bp):
    row = indices[0]
    col = indices[1]
    idpad = 8 * NW * ((NN + 8 * NW - 1) // (8 * NW))
    idx_id = jnp.pad(idx_identity, (0, idpad - NN))
    zeros16 = jnp.zeros((NN, 16), jnp.float32)

    cnts = _sc_scatter_ones(row, col, zeros16)
    row_cnt = cnts[0, :, :1]
    col_cnt = cnts[1, :, :1]

    sums_v = _sc_scatter16(values, row, col, zeros16)
    rowsum = [sums_v[0]]
    colsum = [sums_v[1]]
    gsum = jnp.sum(sums_v[0], axis=0)
    m = jnp.zeros((16,), jnp.float32)
    s = jnp.ones((16,), jnp.float32)

    a_blocks = [values]
    for (W, b) in ((W0, b0), (W1, b1), (W2, b2)):
        C = W.shape[2]
        nb_in = len(a_blocks)
        nb = C // 16
        inv_s = 1.0 / s
        Wf = W * inv_s[None, :, None]
        const = (b + ((gsum / NNZ - m) * inv_s) @ W[4]
                 - (m * inv_s) @ (W[0] + W[1] + W[2] + W[3] + W[5]))

        g = [_sc_gather(ab, idx_transpose) for ab in a_blocks]
        d = [_sc_gather(ab, idx_id)[:NN] for ab in a_blocks]

        def wrows(wi, j):
            return Wf[wi][j * 16:(j + 1) * 16]

        terms_p = ([(a_blocks[j], wrows(0, j), False) for j in range(nb_in)]
                   + [(g[j], wrows(1, j), False) for j in range(nb_in)])
        p_blocks = _tc_matmul(terms_p, n_out=nb)
        if nb == 1:
            p_blocks = [p_blocks]

        terms_r = ([(rowsum[j], wrows(2, j), True) for j in range(nb_in)]
                   + [(d[j], wrows(5, j), False) for j in range(nb_in)])
        rtab = _tc_matmul(terms_r, cnt=row_cnt)
        terms_c = [(colsum[j], wrows(3, j), True) for j in range(nb_in)]
        ctab = _tc_matmul(terms_c, cnt=col_cnt)

        a_blocks, parts = _sc_combine(p_blocks, rtab, ctab, row, col, const)

        sums2 = [_sc_scatter16(ab, row, col, zeros16) for ab in a_blocks]
        rowsum = [s2[0] for s2 in sums2]
        colsum = [s2[1] for s2 in sums2]
        pp = jnp.sum(parts, axis=(0, 1))
        ssum, ssq = pp[:C], pp[C:]
        m = ssum / NNZ
        s = jnp.sqrt(jnp.maximum(ssq / NNZ - m * m, 0.0)) + 1e-5
        gsum = ssum

    inv_s = 1.0 / s
    wp_s = Wp[:, 0] * inv_s
    wp_pad = jnp.zeros((16, 128, 2), jnp.float32)
    wp_pad = wp_pad.at[:, 0, :].set(wp_s.reshape(2, 16).T)
    cp = jnp.zeros((1, 128), jnp.float32)
    cp = cp.at[0, 0].set(bp[0] - jnp.dot(m * inv_s, Wp[:, 0]))
    terms_f = [(rowsum[j], wp_pad[:, :, j], True) for j in range(len(rowsum))]
    out = _tc_matmul(terms_f, cnt=row_cnt, const=cp)
    return out[:, :1]


# speedup vs baseline: 3.2248x; 1.0902x over previous
import functools

import jax
import jax.numpy as jnp
from jax import lax
from jax.experimental import pallas as pl
from jax.experimental.pallas import tpu as pltpu
from jax.experimental.pallas import tpu_sc as plsc

NN = 50000
NNZ = 800000
NC, NS, LANES = 2, 16, 16
NW = NC * NS

_MESH = dict(core_axis_name="c", subcore_axis_name="s")
_NOTILE = pltpu.CompilerParams(use_tc_tiling_on_sc=False)


def _pipe2(nch, issue, process, carry0=0):
    issue(0, 0)
    if nch == 1:
        return process(0, 0, carry0)
    odd = nch % 2 == 1
    npairs = (nch - 1) // 2 if odd else nch // 2

    def pair(p, carry):
        i0 = 2 * p
        issue(i0 + 1, 1)
        carry = process(i0, 0, carry)
        if odd:
            issue(i0 + 2, 0)
        else:
            @pl.when(i0 + 2 < nch)
            def _():
                issue(i0 + 2, 0)
        carry = process(i0 + 1, 1, carry)
        return carry

    carry = lax.fori_loop(0, npairs, pair, carry0)
    if odd:
        carry = process(nch - 1, 0, carry)
    return carry


def _tc_matmul(terms, cnt=None, const=None, n_out=1, blk=5000):
    M = terms[0][0].shape[0]
    N = terms[0][1].shape[1]
    assert M % blk == 0
    nt = len(terms)
    hasc = cnt is not None
    hask = const is not None

    def body(*refs):
        i = 2 * nt
        cnt_ref = k_ref = None
        if hasc:
            cnt_ref = refs[i]; i += 1
        if hask:
            k_ref = refs[i]; i += 1
        acc = None
        inv = None
        if hasc:
            inv = 1.0 / jnp.maximum(cnt_ref[...], 1.0)
        for t in range(nt):
            x = refs[2 * t][...]
            if terms[t][2]:
                x = x * inv
            p = jnp.dot(x, refs[2 * t + 1][...],
                        preferred_element_type=jnp.float32)
            acc = p if acc is None else acc + p
        if hask:
            acc = acc + k_ref[...]
        h = N // n_out
        for o in range(n_out):
            refs[i + o][...] = acc[:, o * h:(o + 1) * h]

    in_specs = []
    args = []
    for (x, w, _) in terms:
        K = x.shape[1]
        in_specs += [pl.BlockSpec((blk, K), lambda i: (i, 0)),
                     pl.BlockSpec((K, N), lambda i: (0, 0))]
        args += [x, w]
    if hasc:
        in_specs += [pl.BlockSpec((blk, 1), lambda i: (i, 0))]
        args += [cnt]
    if hask:
        in_specs += [pl.BlockSpec((1, N), lambda i: (0, 0))]
        args += [const]

    h = N // n_out
    out_specs = [pl.BlockSpec((blk, h), lambda i: (i, 0))] * n_out
    out_shape = [jax.ShapeDtypeStruct((M, h), jnp.float32)] * n_out
    if n_out == 1:
        out_specs, out_shape = out_specs[0], out_shape[0]

    return pl.pallas_call(
        body,
        grid=(M // blk,),
        in_specs=in_specs,
        out_specs=out_specs,
        out_shape=out_shape,
        compiler_params=pltpu.CompilerParams(
            vmem_limit_bytes=100 * 1024 * 1024),
    )(*args)


def _sc_gather_layer(tables, idx_t, idx_id):
    nb = len(tables)
    M = idx_t.shape[0]
    Mi = idx_id.shape[0]
    epw = M // NW
    epw_i = Mi // NW
    ch = 1000 if nb <= 2 else 200
    seg = 5000
    ch_i = 224
    rows = max(ch, ch_i)

    out_type = tuple([jax.ShapeDtypeStruct((M, 16), jnp.float32)] * nb
                     + [jax.ShapeDtypeStruct((Mi, 16), jnp.float32)] * nb)
    scratch = ([pltpu.VMEM((seg,), jnp.int32)]
               + [pltpu.VMEM((rows, 16), jnp.float32)
                  for _ in range(2 * nb)]
               + [pltpu.SemaphoreType.DMA, pltpu.SemaphoreType.DMA])

    @functools.partial(
        pl.kernel,
        out_type=out_type,
        mesh=plsc.VectorSubcoreMesh(**_MESH),
        compiler_params=_NOTILE,
        scratch_types=scratch,
    )
    def k(*refs):
        i = 0
        tab_h = refs[i:i + nb]; i += nb
        idxt_h = refs[i]; i += 1
        idxi_h = refs[i]; i += 1
        g_h = refs[i:i + nb]; i += nb
        d_h = refs[i:i + nb]; i += nb
        idx_v = refs[i]; i += 1
        bufs = [refs[i:i + nb], refs[i + nb:i + 2 * nb]]; i += 2 * nb
        sems = refs[i:i + 2]

        wid = lax.axis_index("s") * NC + lax.axis_index("c")

        def phase(idx_h, out_hs, pepw, pseg, pch):
            nseg = pepw // pseg
            nchp = pseg // pch
            base_w = wid * pepw
            for si in range(nseg):
                seg_base = base_w + si * pseg
                pltpu.sync_copy(idx_h.at[pl.ds(seg_base, pseg)],
                                idx_v.at[pl.ds(0, pseg)])

                def issue(ci, slot):
                    isl = idx_v.at[pl.ds(ci * pch, pch)]
                    for b in range(nb):
                        pltpu.async_copy(tab_h[b].at[isl],
                                         bufs[slot][b].at[pl.ds(0, pch)],
                                         sems[slot])

                def process(ci, slot, carry):
                    isl = idx_v.at[pl.ds(ci * pch, pch)]
                    base = seg_base + ci * pch
                    for b in range(nb):
                        pltpu.make_async_copy(
                            tab_h[b].at[isl],
                            bufs[slot][b].at[pl.ds(0, pch)],
                            sems[slot]).wait()
                    for b in range(nb):
                        pltpu.sync_copy(bufs[slot][b].at[pl.ds(0, pch)],
                                        out_hs[b].at[pl.ds(base, pch)])
                    return carry

                _pipe2(nchp, issue, process)

        phase(idxt_h, g_h, epw, seg, ch)
        phase(idxi_h, d_h, epw_i, epw_i, ch_i)

    res = k(*tables, idx_t, idx_id)
    return list(res[:nb]), list(res[nb:])


def _flush_ranges(body):
    s = lax.axis_index("s")

    @pl.when(s < 15)
    def _():
        body(s * 3128, 3128)

    @pl.when(s == 15)
    def _():
        body(15 * 3128, NN - 15 * 3128)


def _sc_scatter(a_block, row, col, zeros):
    M = row.shape[0]
    ept = M // NS
    ch = 2000
    nch = ept // ch

    @functools.partial(
        pl.kernel,
        out_type=jax.ShapeDtypeStruct((2, NN, 16), jnp.float32),
        mesh=plsc.VectorSubcoreMesh(**_MESH),
        compiler_params=_NOTILE,
        scratch_types=[
            pltpu.VMEM((ch,), jnp.int32), pltpu.VMEM((ch,), jnp.int32),
            pltpu.VMEM((ch, 16), jnp.float32),
            pltpu.VMEM((ch, 16), jnp.float32),
            pltpu.VMEM_SHARED((NN, 16), jnp.float32),
            pltpu.SemaphoreType.DMA, pltpu.SemaphoreType.DMA,
        ],
    )
    def k(a_h, row_h, col_h, z_h, out_h, idx0, idx1, dat0, dat1, accum,
          sem0, sem1):
        idx_v = [idx0, idx1]
        data_v = [dat0, dat1]
        sems = [sem0, sem1]
        c = lax.axis_index("c")
        s = lax.axis_index("s")
        _flush_ranges(lambda bb, n: pltpu.sync_copy(
            z_h.at[pl.ds(bb, n)], accum.at[pl.ds(bb, n)]))
        plsc.subcore_barrier()

        def issue(ci, slot):
            base = s * ept + ci * ch

            @pl.when(c == 0)
            def _():
                pltpu.async_copy(row_h.at[pl.ds(base, ch)],
                                 idx_v[slot], sems[slot])

            @pl.when(c == 1)
            def _():
                pltpu.async_copy(col_h.at[pl.ds(base, ch)],
                                 idx_v[slot], sems[slot])

            pltpu.async_copy(a_h.at[pl.ds(base, ch)], data_v[slot],
                             sems[slot])

        def process(ci, slot, carry):
            base = s * ept + ci * ch
            pltpu.make_async_copy(row_h.at[pl.ds(base, ch)],
                                  idx_v[slot], sems[slot]).wait()
            pltpu.make_async_copy(a_h.at[pl.ds(base, ch)],
                                  data_v[slot], sems[slot]).wait()
            pltpu.sync_copy(data_v[slot], accum.at[idx_v[slot]], add=True)
            return carry

        _pipe2(nch, issue, process)
        plsc.subcore_barrier()
        _flush_ranges(lambda bb, n: pltpu.sync_copy(
            accum.at[pl.ds(bb, n)], out_h.at[c, pl.ds(bb, n)]))

    return k(a_block, row, col, zeros)


def _sc_scatter_ones(row, col, zeros):
    M = row.shape[0]
    ept = M // NS
    ch = 2000
    nch = ept // ch

    @functools.partial(
        pl.kernel,
        out_type=jax.ShapeDtypeStruct((2, NN, 16), jnp.float32),
        mesh=plsc.VectorSubcoreMesh(**_MESH),
        compiler_params=_NOTILE,
        scratch_types=[
            pltpu.VMEM((ch,), jnp.int32), pltpu.VMEM((ch,), jnp.int32),
            pltpu.VMEM((ch, 16), jnp.float32),
            pltpu.VMEM_SHARED((NN, 16), jnp.float32),
            pltpu.SemaphoreType.DMA, pltpu.SemaphoreType.DMA,
        ],
    )
    def k(row_h, col_h, z_h, out_h, idx0, idx1, ones_v, accum, sem0, sem1):
        idx_v = [idx0, idx1]
        sems = [sem0, sem1]
        c = lax.axis_index("c")
        s = lax.axis_index("s")
        one = jnp.full((16,), 1.0, jnp.float32)

        def fill(j, carry):
            ones_v[j, :] = one
            return carry

        lax.fori_loop(0, ch, fill, 0)
        _flush_ranges(lambda bb, n: pltpu.sync_copy(
            z_h.at[pl.ds(bb, n)], accum.at[pl.ds(bb, n)]))
        plsc.subcore_barrier()

        def issue(ci, slot):
            base = s * ept + ci * ch

            @pl.when(c == 0)
            def _():
                pltpu.async_copy(row_h.at[pl.ds(base, ch)],
                                 idx_v[slot], sems[slot])

            @pl.when(c == 1)
            def _():
                pltpu.async_copy(col_h.at[pl.ds(base, ch)],
                                 idx_v[slot], sems[slot])

        def process(ci, slot, carry):
            base = s * ept + ci * ch
            pltpu.make_async_copy(row_h.at[pl.ds(base, ch)],
                                  idx_v[slot], sems[slot]).wait()
            pltpu.sync_copy(ones_v, accum.at[idx_v[slot]], add=True)
            return carry

        _pipe2(nch, issue, process)
        plsc.subcore_barrier()
        _flush_ranges(lambda bb, n: pltpu.sync_copy(
            accum.at[pl.ds(bb, n)], out_h.at[c, pl.ds(bb, n)]))

    return k(row, col, zeros)


def _sc_combine(p_blocks, rtab, ctab, row, col, const):
    nb = len(p_blocks)
    M = p_blocks[0].shape[0]
    C = rtab.shape[1]
    assert C == 16 * nb
    epw = M // NW
    ch = 200
    seg = 5000
    nseg = epw // seg
    nchp = seg // ch

    out_type = tuple([jax.ShapeDtypeStruct((M, 16), jnp.float32)] * nb
                     + [jax.ShapeDtypeStruct((NW, 1, 2 * C), jnp.float32)])
    scratch = ([pltpu.VMEM((seg,), jnp.int32), pltpu.VMEM((seg,), jnp.int32)]
               + [pltpu.VMEM((ch, 16), jnp.float32) for _ in range(2 * nb)]
               + [pltpu.VMEM((ch, C), jnp.float32) for _ in range(4)]
               + [pltpu.VMEM((C,), jnp.float32),
                  pltpu.VMEM((1, 2 * C), jnp.float32),
                  pltpu.SemaphoreType.DMA, pltpu.SemaphoreType.DMA])

    @functools.partial(
        pl.kernel,
        out_type=out_type,
        mesh=plsc.VectorSubcoreMesh(**_MESH),
        compiler_params=_NOTILE,
        scratch_types=scratch,
    )
    def k(*refs):
        i = 0
        p_h = refs[i:i + nb]; i += nb
        r_h = refs[i]; i += 1
        c_h = refs[i]; i += 1
        row_h = refs[i]; i += 1
        col_h = refs[i]; i += 1
        const_h = refs[i]; i += 1
        a_h = refs[i:i + nb]; i += nb
        part_h = refs[i]; i += 1
        ri = refs[i]; i += 1
        ci_v = refs[i]; i += 1
        p_v = [refs[i:i + nb], refs[i + nb:i + 2 * nb]]; i += 2 * nb
        rg = refs[i:i + 2]; i += 2
        cg = refs[i + 0:i + 2]; i += 2
        const_v = refs[i]; i += 1
        part_v = refs[i]; i += 1
        sems = refs[i:i + 2]

        wid = lax.axis_index("s") * NC + lax.axis_index("c")
        base_w = wid * epw
        pltpu.sync_copy(const_h, const_v)
        cvecs = [const_v[pl.ds(cb * 16, 16)] for cb in range(nb)]
        zero = jnp.zeros((16,), jnp.float32)

        sums = tuple(zero for _ in range(2 * nb))
        for si in range(nseg):
            seg_base = base_w + si * seg
            pltpu.sync_copy(row_h.at[pl.ds(seg_base, seg)], ri)
            pltpu.sync_copy(col_h.at[pl.ds(seg_base, seg)], ci_v)

            def issue(cii, slot):
                base = seg_base + cii * ch
                for b in range(nb):
                    pltpu.async_copy(p_h[b].at[pl.ds(base, ch)],
                                     p_v[slot][b], sems[slot])
                pltpu.async_copy(r_h.at[ri.at[pl.ds(cii * ch, ch)]],
                                 rg[slot], sems[slot])
                pltpu.async_copy(c_h.at[ci_v.at[pl.ds(cii * ch, ch)]],
                                 cg[slot], sems[slot])

            def process(cii, slot, sums):
                base = seg_base + cii * ch
                for b in range(nb):
                    pltpu.make_async_copy(p_h[b].at[pl.ds(base, ch)],
                                          p_v[slot][b], sems[slot]).wait()
                pltpu.make_async_copy(r_h.at[ri.at[pl.ds(cii * ch, ch)]],
                                      rg[slot], sems[slot]).wait()
                pltpu.make_async_copy(c_h.at[ci_v.at[pl.ds(cii * ch, ch)]],
                                      cg[slot], sems[slot]).wait()

                def rowloop(j, sums):
                    out = []
                    for cb in range(nb):
                        sl = pl.ds(cb * 16, 16)
                        x = (p_v[slot][cb][j, :] + rg[slot][j, sl]
                             + cg[slot][j, sl] + cvecs[cb])
                        x = jnp.maximum(x, 0.0)
                        p_v[slot][cb][j, :] = x
                        out.append(sums[2 * cb] + x)
                        out.append(sums[2 * cb + 1] + x * x)
                    return tuple(out)

                sums = lax.fori_loop(0, ch, rowloop, sums)
                for b in range(nb):
                    pltpu.sync_copy(p_v[slot][b], a_h[b].at[pl.ds(base, ch)])
                return sums

            sums = _pipe2(nchp, issue, process, sums)

        for cb in range(nb):
            part_v[0, pl.ds(cb * 16, 16)] = sums[2 * cb]
            part_v[0, pl.ds(C + cb * 16, 16)] = sums[2 * cb + 1]
        pltpu.sync_copy(part_v, part_h.at[wid])

    res = k(*p_blocks, rtab, ctab, row, col, const)
    return list(res[:nb]), res[nb]


def kernel(values, indices, idx_identity, idx_transpose,
           W0, b0, W1, b1, W2, b2, Wp, bp):
    row = indices[0]
    col = indices[1]
    idpad = 8 * NW * ((NN + 8 * NW - 1) // (8 * NW))
    idx_id = jnp.pad(idx_identity, (0, idpad - NN))
    zeros16 = jnp.zeros((NN, 16), jnp.float32)

    cnts = _sc_scatter_ones(row, col, zeros16)
    row_cnt = cnts[0, :, :1]
    col_cnt = cnts[1, :, :1]
    sums_v = _sc_scatter(values, row, col, zeros16)
    rowsum = [sums_v[0]]
    colsum = [sums_v[1]]
    gsum = jnp.sum(sums_v[0], axis=0)
    m = jnp.zeros((16,), jnp.float32)
    s = jnp.ones((16,), jnp.float32)

    a_blocks = [values]
    for (W, b) in ((W0, b0), (W1, b1), (W2, b2)):
        C = W.shape[2]
        nb_in = len(a_blocks)
        nb = C // 16
        inv_s = 1.0 / s
        Wf = W * inv_s[None, :, None]
        const = (b + ((gsum / NNZ - m) * inv_s) @ W[4]
                 - (m * inv_s) @ (W[0] + W[1] + W[2] + W[3] + W[5]))

        g, d = _sc_gather_layer(a_blocks, idx_transpose, idx_id)
        d = [db[:NN] for db in d]

        def wrows(wi, j):
            return Wf[wi][j * 16:(j + 1) * 16]

        terms_p = ([(a_blocks[j], wrows(0, j), False) for j in range(nb_in)]
                   + [(g[j], wrows(1, j), False) for j in range(nb_in)])
        p_blocks = _tc_matmul(terms_p, n_out=nb)
        if nb == 1:
            p_blocks = [p_blocks]

        terms_r = ([(rowsum[j], wrows(2, j), True) for j in range(nb_in)]
                   + [(d[j], wrows(5, j), False) for j in range(nb_in)])
        rtab = _tc_matmul(terms_r, cnt=row_cnt)
        terms_c = [(colsum[j], wrows(3, j), True) for j in range(nb_in)]
        ctab = _tc_matmul(terms_c, cnt=col_cnt)

        a_blocks, parts = _sc_combine(p_blocks, rtab, ctab, row, col, const)

        sums2 = [_sc_scatter(ab, row, col, zeros16) for ab in a_blocks]
        rowsum = [s2[0] for s2 in sums2]
        colsum = [s2[1] for s2 in sums2]
        pp = jnp.sum(parts, axis=(0, 1))
        ssum, ssq = pp[:C], pp[C:]
        m = ssum / NNZ
        s = jnp.sqrt(jnp.maximum(ssq / NNZ - m * m, 0.0)) + 1e-5
        gsum = ssum

    inv_s = 1.0 / s
    wp_s = Wp[:, 0] * inv_s
    wp_pad = jnp.zeros((16, 128, 2), jnp.float32)
    wp_pad = wp_pad.at[:, 0, :].set(wp_s.reshape(2, 16).T)
    cp = jnp.zeros((1, 128), jnp.float32)
    cp = cp.at[0, 0].set(bp[0] - jnp.dot(m * inv_s, Wp[:, 0]))
    terms_f = [(rowsum[j], wp_pad[:, :, j], True) for j in range(len(rowsum))]
    out = _tc_matmul(terms_f, cnt=row_cnt, const=cp)
    return out[:, :1]


# speedup vs baseline: 9.1924x; 2.8505x over previous
import functools

import jax
import jax.numpy as jnp
from jax import lax
from jax.experimental import pallas as pl
from jax.experimental.pallas import tpu as pltpu
from jax.experimental.pallas import tpu_sc as plsc

NN = 50000
NNZ = 800000
NC, NS, LANES = 2, 16, 16
NW = NC * NS

_MESH = dict(core_axis_name="c", subcore_axis_name="s")
_NOTILE = pltpu.CompilerParams(use_tc_tiling_on_sc=False)


def _pipe2(nch, issue, process, carry0=0):
    issue(0, 0)
    if nch == 1:
        return process(0, 0, carry0)
    odd = nch % 2 == 1
    npairs = (nch - 1) // 2 if odd else nch // 2

    def pair(p, carry):
        i0 = 2 * p
        issue(i0 + 1, 1)
        carry = process(i0, 0, carry)
        if odd:
            issue(i0 + 2, 0)
        else:
            @pl.when(i0 + 2 < nch)
            def _():
                issue(i0 + 2, 0)
        carry = process(i0 + 1, 1, carry)
        return carry

    carry = lax.fori_loop(0, npairs, pair, carry0)
    if odd:
        carry = process(nch - 1, 0, carry)
    return carry


def _tc_matmul(terms, cnt=None, const=None, n_out=1, blk=5000):
    M = terms[0][0].shape[0]
    N = terms[0][1].shape[1]
    assert M % blk == 0
    nt = len(terms)
    hasc = cnt is not None
    hask = const is not None

    def body(*refs):
        i = 2 * nt
        cnt_ref = k_ref = None
        if hasc:
            cnt_ref = refs[i]; i += 1
        if hask:
            k_ref = refs[i]; i += 1
        acc = None
        inv = None
        if hasc:
            inv = 1.0 / jnp.maximum(cnt_ref[...], 1.0)
        for t in range(nt):
            x = refs[2 * t][...]
            if terms[t][2]:
                x = x * inv
            p = jnp.dot(x, refs[2 * t + 1][...],
                        preferred_element_type=jnp.float32)
            acc = p if acc is None else acc + p
        if hask:
            acc = acc + k_ref[...]
        h = N // n_out
        for o in range(n_out):
            refs[i + o][...] = acc[:, o * h:(o + 1) * h]

    in_specs = []
    args = []
    for (x, w, _) in terms:
        K = x.shape[1]
        in_specs += [pl.BlockSpec((blk, K), lambda i: (i, 0)),
                     pl.BlockSpec((K, N), lambda i: (0, 0))]
        args += [x, w]
    if hasc:
        in_specs += [pl.BlockSpec((blk, 1), lambda i: (i, 0))]
        args += [cnt]
    if hask:
        in_specs += [pl.BlockSpec((1, N), lambda i: (0, 0))]
        args += [const]

    h = N // n_out
    out_specs = [pl.BlockSpec((blk, h), lambda i: (i, 0))] * n_out
    out_shape = [jax.ShapeDtypeStruct((M, h), jnp.float32)] * n_out
    if n_out == 1:
        out_specs, out_shape = out_specs[0], out_shape[0]

    return pl.pallas_call(
        body,
        grid=(M // blk,),
        in_specs=in_specs,
        out_specs=out_specs,
        out_shape=out_shape,
        compiler_params=pltpu.CompilerParams(
            vmem_limit_bytes=100 * 1024 * 1024),
    )(*args)


def _tc_edge_matmul(xs, w8, n_out, blk=4000):
    M8 = xs[0].shape[0]
    nj = len(xs)
    assert M8 % blk == 0

    def body(*refs):
        x_refs = refs[:nj]
        w_refs = refs[nj:nj + n_out * nj]
        o_refs = refs[nj + n_out * nj:]
        xs_v = [x[...] for x in x_refs]
        for o in range(n_out):
            acc = None
            for j in range(nj):
                p = jnp.dot(xs_v[j], w_refs[o * nj + j][...],
                            preferred_element_type=jnp.float32)
                acc = p if acc is None else acc + p
            o_refs[o][...] = acc

    in_specs = ([pl.BlockSpec((blk, 128), lambda i: (i, 0))] * nj
                + [pl.BlockSpec((128, 128), lambda i: (0, 0))] * (n_out * nj))
    args = list(xs) + [w8[o][j] for o in range(n_out) for j in range(nj)]
    out_specs = [pl.BlockSpec((blk, 128), lambda i: (i, 0))] * n_out
    out_shape = [jax.ShapeDtypeStruct((M8, 128), jnp.float32)] * n_out

    return pl.pallas_call(
        body,
        grid=(M8 // blk,),
        in_specs=in_specs,
        out_specs=out_specs,
        out_shape=out_shape,
        compiler_params=pltpu.CompilerParams(
            vmem_limit_bytes=100 * 1024 * 1024),
    )(*args)


def _tc_pool(xs, wrows, cnt, const_s, blk=5000):
    M = xs[0].shape[0]
    nj = len(xs)

    def body(*refs):
        x_refs = refs[:nj]
        w_refs = refs[nj:2 * nj]
        cnt_ref = refs[2 * nj]
        k_ref = refs[2 * nj + 1]
        o_ref = refs[2 * nj + 2]
        inv = 1.0 / jnp.maximum(cnt_ref[...], 1.0)
        acc = None
        for j in range(nj):
            p = jnp.sum(x_refs[j][...] * w_refs[j][...], axis=1,
                        keepdims=True)
            acc = p if acc is None else acc + p
        acc = acc * inv + k_ref[...]
        o_ref[...] = jnp.broadcast_to(acc, (acc.shape[0], 8))

    in_specs = ([pl.BlockSpec((blk, 16), lambda i: (i, 0))] * nj
                + [pl.BlockSpec((1, 16), lambda i: (0, 0))] * nj
                + [pl.BlockSpec((blk, 1), lambda i: (i, 0)),
                   pl.BlockSpec((1, 1), lambda i: (0, 0))])
    args = list(xs) + list(wrows) + [cnt, const_s]
    return pl.pallas_call(
        body,
        grid=(M // blk,),
        in_specs=in_specs,
        out_specs=pl.BlockSpec((blk, 8), lambda i: (i, 0)),
        out_shape=jax.ShapeDtypeStruct((M, 8), jnp.float32),
        compiler_params=pltpu.CompilerParams(
            vmem_limit_bytes=100 * 1024 * 1024),
    )(*args)


def _sc_gather_diag(tables, idx_id):
    nb = len(tables)
    Mi = idx_id.shape[0]
    epw = Mi // NW
    ch = 224
    nch = epw // ch

    out_type = tuple([jax.ShapeDtypeStruct((Mi, 16), jnp.float32)] * nb)
    if nb == 1:
        out_type = out_type[0]
    scratch = ([pltpu.VMEM((epw,), jnp.int32)]
               + [pltpu.VMEM((ch, 16), jnp.float32) for _ in range(2 * nb)]
               + [pltpu.SemaphoreType.DMA, pltpu.SemaphoreType.DMA])

    @functools.partial(
        pl.kernel,
        out_type=out_type,
        mesh=plsc.VectorSubcoreMesh(**_MESH),
        compiler_params=_NOTILE,
        scratch_types=scratch,
    )
    def k(*refs):
        i = 0
        tab_h = refs[i:i + nb]; i += nb
        idx_h = refs[i]; i += 1
        d_h = refs[i:i + nb]; i += nb
        idx_v = refs[i]; i += 1
        bufs = [refs[i:i + nb], refs[i + nb:i + 2 * nb]]; i += 2 * nb
        sems = refs[i:i + 2]

        wid = lax.axis_index("s") * NC + lax.axis_index("c")
        base_w = wid * epw
        pltpu.sync_copy(idx_h.at[pl.ds(base_w, epw)], idx_v)

        def issue(ci, slot):
            isl = idx_v.at[pl.ds(ci * ch, ch)]
            for b in range(nb):
                pltpu.async_copy(tab_h[b].at[isl], bufs[slot][b], sems[slot])

        def process(ci, slot, carry):
            isl = idx_v.at[pl.ds(ci * ch, ch)]
            base = base_w + ci * ch
            for b in range(nb):
                pltpu.make_async_copy(tab_h[b].at[isl], bufs[slot][b],
                                      sems[slot]).wait()
            for b in range(nb):
                pltpu.sync_copy(bufs[slot][b], d_h[b].at[pl.ds(base, ch)])
            return carry

        _pipe2(nch, issue, process)

    res = k(*tables, idx_id)
    return list(res) if nb > 1 else [res]


def _flush_ranges(body):
    s = lax.axis_index("s")

    @pl.when(s < 15)
    def _():
        body(s * 3128, 3128)

    @pl.when(s == 15)
    def _():
        body(15 * 3128, NN - 15 * 3128)


def _sc_scatter(a_block, row, col, zeros):
    M = row.shape[0]
    ept = M // NS
    ch = 2000
    nch = ept // ch

    @functools.partial(
        pl.kernel,
        out_type=(jax.ShapeDtypeStruct((NN, 16), jnp.float32),
                  jax.ShapeDtypeStruct((NN, 16), jnp.float32)),
        mesh=plsc.VectorSubcoreMesh(**_MESH),
        compiler_params=_NOTILE,
        scratch_types=[
            pltpu.VMEM((ch,), jnp.int32), pltpu.VMEM((ch,), jnp.int32),
            pltpu.VMEM((ch, 16), jnp.float32),
            pltpu.VMEM((ch, 16), jnp.float32),
            pltpu.VMEM_SHARED((NN, 16), jnp.float32),
            pltpu.SemaphoreType.DMA, pltpu.SemaphoreType.DMA,
        ],
    )
    def k(a_h, row_h, col_h, z_h, outr_h, outc_h, idx0, idx1, dat0, dat1,
          accum, sem0, sem1):
        idx_v = [idx0, idx1]
        data_v = [dat0, dat1]
        sems = [sem0, sem1]
        c = lax.axis_index("c")
        s = lax.axis_index("s")
        _flush_ranges(lambda bb, n: pltpu.sync_copy(
            z_h.at[pl.ds(bb, n)], accum.at[pl.ds(bb, n)]))
        plsc.subcore_barrier()

        def issue(ci, slot):
            base = s * ept + ci * ch

            @pl.when(c == 0)
            def _():
                pltpu.async_copy(row_h.at[pl.ds(base, ch)],
                                 idx_v[slot], sems[slot])

            @pl.when(c == 1)
            def _():
                pltpu.async_copy(col_h.at[pl.ds(base, ch)],
                                 idx_v[slot], sems[slot])

            pltpu.async_copy(a_h.at[pl.ds(base, ch)], data_v[slot],
                             sems[slot])

        def process(ci, slot, carry):
            base = s * ept + ci * ch
            pltpu.make_async_copy(row_h.at[pl.ds(base, ch)],
                                  idx_v[slot], sems[slot]).wait()
            pltpu.make_async_copy(a_h.at[pl.ds(base, ch)],
                                  data_v[slot], sems[slot]).wait()
            pltpu.sync_copy(data_v[slot], accum.at[idx_v[slot]], add=True)
            return carry

        _pipe2(nch, issue, process)
        plsc.subcore_barrier()

        def flush(bb, n):
            @pl.when(c == 0)
            def _():
                pltpu.sync_copy(accum.at[pl.ds(bb, n)],
                                outr_h.at[pl.ds(bb, n)])

            @pl.when(c == 1)
            def _():
                pltpu.sync_copy(accum.at[pl.ds(bb, n)],
                                outc_h.at[pl.ds(bb, n)])

        _flush_ranges(flush)

    return k(a_block, row, col, zeros)


def _sc_scatter_ones(row, col, zeros):
    M = row.shape[0]
    ept = M // NS
    ch = 2000
    nch = ept // ch

    @functools.partial(
        pl.kernel,
        out_type=(jax.ShapeDtypeStruct((NN, 16), jnp.float32),
                  jax.ShapeDtypeStruct((NN, 16), jnp.float32)),
        mesh=plsc.VectorSubcoreMesh(**_MESH),
        compiler_params=_NOTILE,
        scratch_types=[
            pltpu.VMEM((ch,), jnp.int32), pltpu.VMEM((ch,), jnp.int32),
            pltpu.VMEM((ch, 16), jnp.float32),
            pltpu.VMEM_SHARED((NN, 16), jnp.float32),
            pltpu.SemaphoreType.DMA, pltpu.SemaphoreType.DMA,
        ],
    )
    def k(row_h, col_h, z_h, outr_h, outc_h, idx0, idx1, ones_v, accum,
          sem0, sem1):
        idx_v = [idx0, idx1]
        sems = [sem0, sem1]
        c = lax.axis_index("c")
        s = lax.axis_index("s")
        one = jnp.full((16,), 1.0, jnp.float32)

        def fill(j, carry):
            ones_v[j, :] = one
            return carry

        lax.fori_loop(0, ch, fill, 0)
        _flush_ranges(lambda bb, n: pltpu.sync_copy(
            z_h.at[pl.ds(bb, n)], accum.at[pl.ds(bb, n)]))
        plsc.subcore_barrier()

        def issue(ci, slot):
            base = s * ept + ci * ch

            @pl.when(c == 0)
            def _():
                pltpu.async_copy(row_h.at[pl.ds(base, ch)],
                                 idx_v[slot], sems[slot])

            @pl.when(c == 1)
            def _():
                pltpu.async_copy(col_h.at[pl.ds(base, ch)],
                                 idx_v[slot], sems[slot])

        def process(ci, slot, carry):
            base = s * ept + ci * ch
            pltpu.make_async_copy(row_h.at[pl.ds(base, ch)],
                                  idx_v[slot], sems[slot]).wait()
            pltpu.sync_copy(ones_v, accum.at[idx_v[slot]], add=True)
            return carry

        _pipe2(nch, issue, process)
        plsc.subcore_barrier()

        def flush(bb, n):
            @pl.when(c == 0)
            def _():
                pltpu.sync_copy(accum.at[pl.ds(bb, n)],
                                outr_h.at[pl.ds(bb, n)])

            @pl.when(c == 1)
            def _():
                pltpu.sync_copy(accum.at[pl.ds(bb, n)],
                                outc_h.at[pl.ds(bb, n)])

        _flush_ranges(flush)

    return k(row, col, zeros)


def _sc_combine(p0_blocks, p1_blocks, rtab, ctab, row, col, tidx, const):
    nb = len(p0_blocks)
    M = p0_blocks[0].shape[0]
    C = rtab.shape[1]
    assert C == 16 * nb
    epw = M // NW
    ch = 200
    seg = 5000
    nseg = epw // seg
    nchp = seg // ch

    out_type = tuple([jax.ShapeDtypeStruct((M, 16), jnp.float32)] * nb
                     + [jax.ShapeDtypeStruct((NW, 1, 2 * C), jnp.float32)])
    scratch = ([pltpu.VMEM((seg,), jnp.int32) for _ in range(3)]
               + [pltpu.VMEM((ch, 16), jnp.float32) for _ in range(4 * nb)]
               + [pltpu.VMEM((ch, C), jnp.float32) for _ in range(4)]
               + [pltpu.VMEM((C,), jnp.float32),
                  pltpu.VMEM((1, 2 * C), jnp.float32),
                  pltpu.SemaphoreType.DMA, pltpu.SemaphoreType.DMA])

    @functools.partial(
        pl.kernel,
        out_type=out_type,
        mesh=plsc.VectorSubcoreMesh(**_MESH),
        compiler_params=_NOTILE,
        scratch_types=scratch,
    )
    def k(*refs):
        i = 0
        p0_h = refs[i:i + nb]; i += nb
        p1_h = refs[i:i + nb]; i += nb
        r_h = refs[i]; i += 1
        c_h = refs[i]; i += 1
        row_h = refs[i]; i += 1
        col_h = refs[i]; i += 1
        tidx_h = refs[i]; i += 1
        const_h = refs[i]; i += 1
        a_h = refs[i:i + nb]; i += nb
        part_h = refs[i]; i += 1
        ri = refs[i]; i += 1
        ci_v = refs[i]; i += 1
        ti_v = refs[i]; i += 1
        p0_v = [refs[i:i + nb], refs[i + nb:i + 2 * nb]]; i += 2 * nb
        p1_v = [refs[i:i + nb], refs[i + nb:i + 2 * nb]]; i += 2 * nb
        rg = refs[i:i + 2]; i += 2
        cg = refs[i:i + 2]; i += 2
        const_v = refs[i]; i += 1
        part_v = refs[i]; i += 1
        sems = refs[i:i + 2]

        wid = lax.axis_index("s") * NC + lax.axis_index("c")
        base_w = wid * epw
        pltpu.sync_copy(const_h, const_v)
        cvecs = [const_v[pl.ds(cb * 16, 16)] for cb in range(nb)]
        zero = jnp.zeros((16,), jnp.float32)

        sums = tuple(zero for _ in range(2 * nb))
        for si in range(nseg):
            seg_base = base_w + si * seg
            pltpu.sync_copy(row_h.at[pl.ds(seg_base, seg)], ri)
            pltpu.sync_copy(col_h.at[pl.ds(seg_base, seg)], ci_v)
            pltpu.sync_copy(tidx_h.at[pl.ds(seg_base, seg)], ti_v)

            def issue(cii, slot):
                base = seg_base + cii * ch
                tsl = ti_v.at[pl.ds(cii * ch, ch)]
                for b in range(nb):
                    pltpu.async_copy(p0_h[b].at[pl.ds(base, ch)],
                                     p0_v[slot][b], sems[slot])
                    pltpu.async_copy(p1_h[b].at[tsl], p1_v[slot][b],
                                     sems[slot])
                pltpu.async_copy(r_h.at[ri.at[pl.ds(cii * ch, ch)]],
                                 rg[slot], sems[slot])
                pltpu.async_copy(c_h.at[ci_v.at[pl.ds(cii * ch, ch)]],
                                 cg[slot], sems[slot])

            def process(cii, slot, sums):
                base = seg_base + cii * ch
                tsl = ti_v.at[pl.ds(cii * ch, ch)]
                for b in range(nb):
                    pltpu.make_async_copy(p0_h[b].at[pl.ds(base, ch)],
                                          p0_v[slot][b], sems[slot]).wait()
                    pltpu.make_async_copy(p1_h[b].at[tsl], p1_v[slot][b],
                                          sems[slot]).wait()
                pltpu.make_async_copy(r_h.at[ri.at[pl.ds(cii * ch, ch)]],
                                      rg[slot], sems[slot]).wait()
                pltpu.make_async_copy(c_h.at[ci_v.at[pl.ds(cii * ch, ch)]],
                                      cg[slot], sems[slot]).wait()

                def rowloop(j, sums):
                    out = []
                    for cb in range(nb):
                        sl = pl.ds(cb * 16, 16)
                        x = (p0_v[slot][cb][j, :] + p1_v[slot][cb][j, :]
                             + rg[slot][j, sl] + cg[slot][j, sl] + cvecs[cb])
                        x = jnp.maximum(x, 0.0)
                        p0_v[slot][cb][j, :] = x
                        out.append(sums[2 * cb] + x)
                        out.append(sums[2 * cb + 1] + x * x)
                    return tuple(out)

                sums = lax.fori_loop(0, ch, rowloop, sums)
                for b in range(nb):
                    pltpu.sync_copy(p0_v[slot][b], a_h[b].at[pl.ds(base, ch)])
                return sums

            sums = _pipe2(nchp, issue, process, sums)

        for cb in range(nb):
            part_v[0, pl.ds(cb * 16, 16)] = sums[2 * cb]
            part_v[0, pl.ds(C + cb * 16, 16)] = sums[2 * cb + 1]
        pltpu.sync_copy(part_v, part_h.at[wid])

    res = k(*p0_blocks, *p1_blocks, rtab, ctab, row, col, tidx, const)
    return list(res[:nb]), res[nb]


def kernel(values, indices, idx_identity, idx_transpose,
           W0, b0, W1, b1, W2, b2, Wp, bp):
    row = indices[0]
    col = indices[1]
    idpad = 8 * NW * ((NN + 8 * NW - 1) // (8 * NW))
    idx_id = jnp.pad(idx_identity, (0, idpad - NN))
    zeros16 = jnp.zeros((NN, 16), jnp.float32)
    eye8 = jnp.eye(8, dtype=jnp.float32)

    def pack(x):
        return jnp.reshape(x, (x.shape[0] // 8, 128))

    def unpack(x, n):
        return jnp.reshape(x, (n, 16))

    cnt_r, cnt_c = _sc_scatter_ones(row, col, zeros16)
    row_cnt = cnt_r[:, :1]
    col_cnt = cnt_c[:, :1]
    rs0, cs0 = _sc_scatter(values, row, col, zeros16)
    rowsum = [rs0]
    colsum = [cs0]
    gsum = jnp.sum(rs0, axis=0)
    m = jnp.zeros((16,), jnp.float32)
    s = jnp.ones((16,), jnp.float32)

    a_blocks = [values]
    for (W, b) in ((W0, b0), (W1, b1), (W2, b2)):
        C = W.shape[2]
        nb_in = len(a_blocks)
        nb = C // 16
        inv_s = 1.0 / s
        Wf = W * inv_s[None, :, None]
        const = (b + ((gsum / NNZ - m) * inv_s) @ W[4]
                 - (m * inv_s) @ (W[0] + W[1] + W[2] + W[3] + W[5]))

        d = _sc_gather_diag(a_blocks, idx_id)
        d = [db[:NN] for db in d]

        def wsub(wi, j, o):
            return Wf[wi][j * 16:(j + 1) * 16, o * 16:(o + 1) * 16]

        xs = [pack(ab) for ab in a_blocks]
        w8 = ([[jnp.kron(eye8, wsub(0, j, o)) for j in range(nb_in)]
               for o in range(nb)]
              + [[jnp.kron(eye8, wsub(1, j, o)) for j in range(nb_in)]
                 for o in range(nb)])
        pk = _tc_edge_matmul(xs, w8, 2 * nb)
        p0_blocks = [unpack(pk[o], NNZ) for o in range(nb)]
        p1_blocks = [unpack(pk[nb + o], NNZ) for o in range(nb)]

        def wrows(wi, j):
            return Wf[wi][j * 16:(j + 1) * 16]

        terms_r = ([(rowsum[j], wrows(2, j), True) for j in range(nb_in)]
                   + [(d[j], wrows(5, j), False) for j in range(nb_in)])
        rtab = _tc_matmul(terms_r, cnt=row_cnt)
        terms_c = [(colsum[j], wrows(3, j), True) for j in range(nb_in)]
        ctab = _tc_matmul(terms_c, cnt=col_cnt)

        a_blocks, parts = _sc_combine(p0_blocks, p1_blocks, rtab, ctab,
                                      row, col, idx_transpose, const)

        sums2 = [_sc_scatter(ab, row, col, zeros16) for ab in a_blocks]
        rowsum = [s2[0] for s2 in sums2]
        colsum = [s2[1] for s2 in sums2]
        pp = jnp.sum(parts, axis=(0, 1))
        ssum, ssq = pp[:C], pp[C:]
        m = ssum / NNZ
        s = jnp.sqrt(jnp.maximum(ssq / NNZ - m * m, 0.0)) + 1e-5
        gsum = ssum

    inv_s = 1.0 / s
    wp_s = Wp[:, 0] * inv_s
    wrows = [wp_s[j * 16:(j + 1) * 16][None, :] for j in range(len(rowsum))]
    cp = (bp[0] - jnp.dot(m * inv_s, Wp[:, 0])).reshape(1, 1)
    out = _tc_pool(rowsum, wrows, row_cnt, cp)
    return out[:, :1]
